# baseline (device time: 27165 ns/iter reference)
import jax
import jax.numpy as jnp
from jax import lax
from jax.experimental import pallas as pl
from jax.experimental.pallas import tpu as pltpu

N_DEV = 16
B, SQ, SKV, HQ_LOC, DH, DM = 2, 256, 256, 4, 64, 512
ROWS = B * SQ
F_LOC = HQ_LOC * DH
CHUNK = ROWS // N_DEV
CPB = SQ // CHUNK

_SEND_ORDER = [8, 9, 7, 10, 6, 11, 5, 12, 4, 13, 3, 14, 2, 15, 1]


def kernel(x, Wq, K_ext, V_ext, Wo):
    my = lax.axis_index("i")
    kh = lax.dynamic_slice_in_dim(K_ext, my * HQ_LOC, HQ_LOC, axis=2)
    vh = lax.dynamic_slice_in_dim(V_ext, my * HQ_LOC, HQ_LOC, axis=2)
    kh = kh.reshape(B * SKV, HQ_LOC * DH)
    vh = vh.reshape(B * SKV, HQ_LOC * DH)
    x2d = x.reshape(ROWS, DM)

    def body(x_ref, wq_ref, kh_ref, vh_ref, wo_ref, out_ref,
             pbf_ref, comm_ref, comm2_ref, red_ref, ctx_ref,
             send1, recv1, send2, recv2):
        my_pos = lax.axis_index("i")

        bar = pltpu.get_barrier_semaphore()
        for d in range(1, N_DEV):
            pl.semaphore_signal(
                bar, inc=1,
                device_id=((my_pos + d) % N_DEV,),
                device_id_type=pl.DeviceIdType.MESH,
            )

        q = jnp.dot(x_ref[...].astype(jnp.bfloat16),
                    wq_ref[...].astype(jnp.bfloat16),
                    preferred_element_type=jnp.float32)
        ri = lax.broadcasted_iota(jnp.int32, (SQ, SKV), 0) // 64
        ci = lax.broadcasted_iota(jnp.int32, (SQ, SKV), 1) // 64
        mask = ci <= ri
        for b in range(B):
            for h in range(HQ_LOC):
                qbh = q[b * SQ:(b + 1) * SQ, h * DH:(h + 1) * DH]
                kbh = kh_ref[b * SKV:(b + 1) * SKV, h * DH:(h + 1) * DH]
                vbh = vh_ref[b * SKV:(b + 1) * SKV, h * DH:(h + 1) * DH]
                s = lax.dot_general(
                    qbh.astype(jnp.bfloat16), kbh.astype(jnp.bfloat16),
                    (((1,), (1,)), ((), ())),
                    preferred_element_type=jnp.float32) * 0.125
                s = jnp.where(mask, s, -1e9)
                w = jnp.exp(s)
                w = w / jnp.sum(w, axis=1, keepdims=True)
                ctx_ref[b * SQ:(b + 1) * SQ, h * DH:(h + 1) * DH] = jnp.dot(
                    w.astype(jnp.bfloat16), vbh.astype(jnp.bfloat16),
                    preferred_element_type=jnp.float32).astype(jnp.bfloat16)
        wo_bf = wo_ref[...].astype(jnp.bfloat16)

        pl.semaphore_wait(bar, N_DEV - 1)


        sends1 = []
        for d in _SEND_ORDER:
            t = (my_pos + d) % N_DEV
            pc = jnp.dot(ctx_ref[pl.ds(t * CHUNK, CHUNK), :], wo_bf,
                         preferred_element_type=jnp.float32)
            pbf_ref[pl.ds(t * CHUNK, CHUNK), :] = pc.astype(jnp.bfloat16)
            rdma = pltpu.make_async_remote_copy(
                src_ref=pbf_ref.at[pl.ds(t * CHUNK, CHUNK), :],
                dst_ref=comm_ref.at[N_DEV - 1 - d],
                send_sem=send1.at[d - 1],
                recv_sem=recv1.at[N_DEV - 1 - d],
                device_id=(t,),
                device_id_type=pl.DeviceIdType.MESH,
            )
            rdma.start()
            sends1.append(rdma)

        accp = jnp.dot(ctx_ref[pl.ds(my_pos * CHUNK, CHUNK), :], wo_bf,
                       preferred_element_type=jnp.float32)
        for k in range(N_DEV - 1):
            rr = pltpu.make_async_remote_copy(
                src_ref=comm_ref.at[k], dst_ref=comm_ref.at[k],
                send_sem=send1.at[k], recv_sem=recv1.at[k],
                device_id=(my_pos,), device_id_type=pl.DeviceIdType.MESH,
            )
            rr.wait_recv()
        acc = accp + jnp.sum(comm_ref[...].astype(jnp.float32), axis=0)
        out_ref[my_pos // CPB, pl.ds((my_pos % CPB) * CHUNK, CHUNK), :] = acc
        red_ref[...] = acc.astype(jnp.bfloat16)

        sends2 = []
        for d in _SEND_ORDER:
            t = (my_pos + d) % N_DEV
            rdma = pltpu.make_async_remote_copy(
                src_ref=red_ref,
                dst_ref=comm2_ref.at[N_DEV - 1 - d],
                send_sem=send2.at[d - 1],
                recv_sem=recv2.at[N_DEV - 1 - d],
                device_id=(t,),
                device_id_type=pl.DeviceIdType.MESH,
            )
            rdma.start()
            sends2.append(rdma)

        for k in range(N_DEV - 1):
            srcdev = (my_pos + k + 1) % N_DEV
            rr = pltpu.make_async_remote_copy(
                src_ref=comm2_ref.at[k], dst_ref=comm2_ref.at[k],
                send_sem=send2.at[k], recv_sem=recv2.at[k],
                device_id=(my_pos,), device_id_type=pl.DeviceIdType.MESH,
            )
            rr.wait_recv()
            out_ref[srcdev // CPB, pl.ds((srcdev % CPB) * CHUNK, CHUNK), :] = (
                comm2_ref[k].astype(jnp.float32))

        for rdma in sends1:
            rdma.wait_send()
        for rdma in sends2:
            rdma.wait_send()

    return pl.pallas_call(
        body,
        out_shape=jax.ShapeDtypeStruct((B, SQ, DM), jnp.float32),
        in_specs=[pl.BlockSpec(memory_space=pltpu.VMEM)] * 5,
        out_specs=pl.BlockSpec(memory_space=pltpu.VMEM),
        scratch_shapes=[
            pltpu.VMEM((ROWS, DM), jnp.bfloat16),
            pltpu.VMEM((N_DEV - 1, CHUNK, DM), jnp.bfloat16),
            pltpu.VMEM((N_DEV - 1, CHUNK, DM), jnp.bfloat16),
            pltpu.VMEM((CHUNK, DM), jnp.bfloat16),
            pltpu.VMEM((ROWS, F_LOC), jnp.bfloat16),
            pltpu.SemaphoreType.DMA((N_DEV - 1,)),
            pltpu.SemaphoreType.DMA((N_DEV - 1,)),
            pltpu.SemaphoreType.DMA((N_DEV - 1,)),
            pltpu.SemaphoreType.DMA((N_DEV - 1,)),
        ],
        compiler_params=pltpu.CompilerParams(collective_id=0),
    )(x2d, Wq, kh, vh, Wo)


# device time: 27129 ns/iter; 1.0013x vs baseline; 1.0013x over previous
import jax
import jax.numpy as jnp
from jax import lax
from jax.experimental import pallas as pl
from jax.experimental.pallas import tpu as pltpu

N_DEV = 16
B, SQ, SKV, HQ_LOC, DH, DM = 2, 256, 256, 4, 64, 512
ROWS = B * SQ
F_LOC = HQ_LOC * DH
CHUNK = ROWS // N_DEV
CPB = SQ // CHUNK

_SEND_ORDER = [8, 9, 7, 10, 6, 11, 5, 12, 4, 13, 3, 14, 2, 15, 1]


def kernel(x, Wq, K_ext, V_ext, Wo):
    my = lax.axis_index("i")
    kh = lax.dynamic_slice_in_dim(K_ext, my * HQ_LOC, HQ_LOC, axis=2)
    vh = lax.dynamic_slice_in_dim(V_ext, my * HQ_LOC, HQ_LOC, axis=2)
    kh = kh.reshape(B * SKV, HQ_LOC * DH)
    vh = vh.reshape(B * SKV, HQ_LOC * DH)
    x2d = x.reshape(ROWS, DM)

    def body(x_ref, wq_ref, kh_ref, vh_ref, wo_ref, out_ref,
             pbf_ref, comm_ref, comm2_ref, red_ref, ctx_ref,
             send1, recv1, send2, recv2):
        my_pos = lax.axis_index("i")

        bar = pltpu.get_barrier_semaphore()
        for d in range(1, N_DEV):
            pl.semaphore_signal(
                bar, inc=1,
                device_id=((my_pos + d) % N_DEV,),
                device_id_type=pl.DeviceIdType.MESH,
            )

        q = jnp.dot(x_ref[...].astype(jnp.bfloat16),
                    wq_ref[...].astype(jnp.bfloat16),
                    preferred_element_type=jnp.float32)
        ri = lax.broadcasted_iota(jnp.int32, (SQ, SKV), 0) // 64
        ci = lax.broadcasted_iota(jnp.int32, (SQ, SKV), 1) // 64
        mask = ci <= ri
        for b in range(B):
            for h in range(HQ_LOC):
                qbh = q[b * SQ:(b + 1) * SQ, h * DH:(h + 1) * DH]
                kbh = kh_ref[b * SKV:(b + 1) * SKV, h * DH:(h + 1) * DH]
                vbh = vh_ref[b * SKV:(b + 1) * SKV, h * DH:(h + 1) * DH]
                s = lax.dot_general(
                    qbh.astype(jnp.bfloat16), kbh.astype(jnp.bfloat16),
                    (((1,), (1,)), ((), ())),
                    preferred_element_type=jnp.float32) * 0.125
                s = jnp.where(mask, s, -1e9)
                w = jnp.exp(s)
                w = w / jnp.sum(w, axis=1, keepdims=True)
                ctx_ref[b * SQ:(b + 1) * SQ, h * DH:(h + 1) * DH] = jnp.dot(
                    w.astype(jnp.bfloat16), vbh.astype(jnp.bfloat16),
                    preferred_element_type=jnp.float32).astype(jnp.bfloat16)
        wo_bf = wo_ref[...].astype(jnp.bfloat16)

        pl.semaphore_wait(bar, N_DEV - 1)


        sends1 = []
        for d in _SEND_ORDER:
            t = (my_pos + d) % N_DEV
            pc = jnp.dot(ctx_ref[pl.ds(t * CHUNK, CHUNK), :], wo_bf,
                         preferred_element_type=jnp.float32)
            pbf_ref[pl.ds(t * CHUNK, CHUNK), :] = pc.astype(jnp.bfloat16)
            rdma = pltpu.make_async_remote_copy(
                src_ref=pbf_ref.at[pl.ds(t * CHUNK, CHUNK), :],
                dst_ref=comm_ref.at[N_DEV - 1 - d],
                send_sem=send1.at[d - 1],
                recv_sem=recv1.at[N_DEV - 1 - d],
                device_id=(t,),
                device_id_type=pl.DeviceIdType.MESH,
            )
            rdma.start()
            sends1.append(rdma)

        accp = jnp.dot(ctx_ref[pl.ds(my_pos * CHUNK, CHUNK), :], wo_bf,
                       preferred_element_type=jnp.float32)
        for k in range(N_DEV - 1):
            rr = pltpu.make_async_remote_copy(
                src_ref=comm_ref.at[k], dst_ref=comm_ref.at[k],
                send_sem=send1.at[k], recv_sem=recv1.at[k],
                device_id=(my_pos,), device_id_type=pl.DeviceIdType.MESH,
            )
            rr.wait_recv()
        acc = accp + jnp.sum(comm_ref[...].astype(jnp.float32), axis=0)
        out_ref[pl.ds(my_pos * CHUNK, CHUNK), :] = acc
        red_ref[...] = acc.astype(jnp.bfloat16)

        sends2 = []
        for d in _SEND_ORDER:
            t = (my_pos + d) % N_DEV
            rdma = pltpu.make_async_remote_copy(
                src_ref=red_ref,
                dst_ref=comm2_ref.at[N_DEV - 1 - d],
                send_sem=send2.at[d - 1],
                recv_sem=recv2.at[N_DEV - 1 - d],
                device_id=(t,),
                device_id_type=pl.DeviceIdType.MESH,
            )
            rdma.start()
            sends2.append(rdma)

        for k in range(N_DEV - 1):
            srcdev = (my_pos + k + 1) % N_DEV
            rr = pltpu.make_async_remote_copy(
                src_ref=comm2_ref.at[k], dst_ref=comm2_ref.at[k],
                send_sem=send2.at[k], recv_sem=recv2.at[k],
                device_id=(my_pos,), device_id_type=pl.DeviceIdType.MESH,
            )
            rr.wait_recv()
            out_ref[pl.ds(srcdev * CHUNK, CHUNK), :] = (
                comm2_ref[k].astype(jnp.float32))

        for rdma in sends1:
            rdma.wait_send()
        for rdma in sends2:
            rdma.wait_send()

    out2d = pl.pallas_call(
        body,
        out_shape=jax.ShapeDtypeStruct((ROWS, DM), jnp.float32),
        in_specs=[pl.BlockSpec(memory_space=pltpu.VMEM)] * 5,
        out_specs=pl.BlockSpec(memory_space=pltpu.VMEM),
        scratch_shapes=[
            pltpu.VMEM((ROWS, DM), jnp.bfloat16),
            pltpu.VMEM((N_DEV - 1, CHUNK, DM), jnp.bfloat16),
            pltpu.VMEM((N_DEV - 1, CHUNK, DM), jnp.bfloat16),
            pltpu.VMEM((CHUNK, DM), jnp.bfloat16),
            pltpu.VMEM((ROWS, F_LOC), jnp.bfloat16),
            pltpu.SemaphoreType.DMA((N_DEV - 1,)),
            pltpu.SemaphoreType.DMA((N_DEV - 1,)),
            pltpu.SemaphoreType.DMA((N_DEV - 1,)),
            pltpu.SemaphoreType.DMA((N_DEV - 1,)),
        ],
        compiler_params=pltpu.CompilerParams(collective_id=0),
    )(x2d, Wq, kh, vh, Wo)
    return out2d.reshape(B, SQ, DM)
